# baseline (speedup 1.0000x reference)
import functools

import jax
import jax.numpy as jnp
from jax import lax
from jax.experimental import pallas as pl
from jax.experimental.pallas import tpu as pltpu
from jax.experimental.pallas import tpu_sc as plsc

_ANCHORS = [
    (1.3221, 1.73145),
    (3.19275, 4.00944),
    (5.05587, 8.09892),
    (9.47112, 4.84053),
    (11.2364, 10.0071),
]
_OBJECT_SCALE = 5.0
_LN2 = 0.6931471805599453
_B, _C, _H, _W = 16, 25, 32, 32
_A = 5


def _sq(x):
    return x * x


def _log_v(x):
    xi = lax.bitcast_convert_type(x, jnp.int32)
    e = (xi >> 23) - 127
    m = lax.bitcast_convert_type((xi & 0x7FFFFF) | (127 << 23), jnp.float32)
    r = (m - 1.0) / (m + 1.0)
    r2 = r * r
    p = r * (2.0 + r2 * (2.0 / 3.0 + r2 * (2.0 / 5.0
                                           + r2 * (2.0 / 7.0 + r2 * (2.0 / 9.0)))))
    return e.astype(jnp.float32) * _LN2 + p


def _corr_body(pred_hbm, tgt_hbm, out_hbm, cell_v, tgt_v, res_v, dsem):
    c = lax.axis_index("c")
    s = lax.axis_index("s")
    b = s

    @pl.when(c == 0)
    def _():
        pltpu.sync_copy(tgt_hbm, tgt_v)
        lane_i = lax.iota(jnp.int32, 16)
        gv = plsc.load_gather(tgt_v, [jnp.full((16,), b, jnp.int32),
                                      lane_i & 3])

        def pick(off):
            return jnp.sum(jnp.where(lane_i == off, gv, 0.0))

        gx = pick(0) * jnp.float32(_W)
        gy = pick(1) * jnp.float32(_H)
        gw = pick(2) * jnp.float32(_W)
        gh = pick(3) * jnp.float32(_H)
        gi = jnp.clip(gx.astype(jnp.int32), 0, _W - 1)
        gj = jnp.clip(gy.astype(jnp.int32), 0, _H - 1)

        def const_vec(vals):
            v = jnp.full((16,), 1.0, dtype=jnp.float32)
            for idx, val in enumerate(vals):
                v = jnp.where(lane_i == idx, jnp.float32(val), v)
            return v

        awv = const_vec([a[0] for a in _ANCHORS])
        ahv = const_vec([a[1] for a in _ANCHORS])
        inter_v = jnp.minimum(awv, gw) * jnp.minimum(ahv, gh)
        union_v = awv * ahv + gw * gh - inter_v
        ratio_v = jnp.where(lane_i < _A, inter_v / union_v, -1.0)
        best_r = jnp.max(ratio_v)
        hit_v = ratio_v == best_r
        best_a = jnp.min(jnp.where(hit_v, lane_i, jnp.int32(99)))
        aw_b = jnp.sum(jnp.where(lane_i == best_a, awv, 0.0))
        ah_b = jnp.sum(jnp.where(lane_i == best_a, ahv, 0.0))

        cb = 5 * best_a
        pltpu.async_copy(pred_hbm.at[b, pl.ds(cb, 5), gj, :], cell_v,
                         dsem).wait()
        goff = (gi >> 4) << 4
        lane = gi - goff

        def cell_val(k):
            v = cell_v[k, pl.ds(goff, 16)]
            return jnp.full((16,), jnp.sum(jnp.where(lane_i == lane, v, 0.0)))

        t0c = cell_val(0)
        t1c = cell_val(1)
        t2c = cell_val(2)
        t3c = cell_val(3)
        t4c = cell_val(4)
        s0c = 1.0 / (1.0 + jnp.exp(-t0c))
        s1c = 1.0 / (1.0 + jnp.exp(-t1c))
        pcc = 1.0 / (1.0 + jnp.exp(-t4c))
        gif = gi.astype(jnp.float32)
        gjf = gj.astype(jnp.float32)
        gxl = gx - gw * 0.5
        gxr = gx + gw * 0.5
        gyl = gy - gh * 0.5
        gyr = gy + gh * 0.5
        garea = gw * gh
        pbx = s0c + gif
        pby = s1c + gjf
        pbw = jnp.exp(t2c) * aw_b
        pbh = jnp.exp(t3c) * ah_b
        cw = (jnp.minimum(pbx + pbw * 0.5, gxr)
              - jnp.maximum(pbx - pbw * 0.5, gxl))
        chh = (jnp.minimum(pby + pbh * 0.5, gyr)
               - jnp.maximum(pby - pbh * 0.5, gyl))
        carea = jnp.maximum(cw, 0.0) * jnp.maximum(chh, 0.0)
        uarea = pbw * pbh + garea - carea
        tconf = carea / uarea
        mstar = jnp.where(tconf > 0.6, 0.0, 1.0)
        lw = _log_v(jnp.full((16,), gw) / jnp.full((16,), aw_b))
        lh = _log_v(jnp.full((16,), gh) / jnp.full((16,), ah_b))
        delta = (_sq(s0c - (gx - gif)) - _sq(s0c - 0.5)
                 + _sq(s1c - (gy - gjf)) - _sq(s1c - 0.5)
                 + _sq(t2c - lw) - t2c * t2c
                 + _sq(t3c - lh) - t3c * t3c
                 + _OBJECT_SCALE * _sq(pcc - tconf) - mstar * pcc * pcc)
        res_v[...] = jnp.where(lane_i == 0, delta, 0.0)
        pltpu.sync_copy(res_v, out_hbm.at[b])


_corr_sc = functools.partial(
    pl.kernel,
    mesh=plsc.VectorSubcoreMesh(core_axis_name="c", subcore_axis_name="s"),
    out_type=jax.ShapeDtypeStruct((_B, 16), jnp.float32),
    compiler_params=pltpu.CompilerParams(needs_layout_passes=False),
    scratch_types=[
        pltpu.VMEM((5, _W), jnp.float32),
        pltpu.VMEM((_B, 4), jnp.float32),
        pltpu.VMEM((16,), jnp.float32),
        pltpu.SemaphoreType.DMA,
    ],
)(_corr_body)


_BPG = 2


def _base_body(tgt_ref, pred_ref, out_ref, acc_ref):
    bidx = pl.program_id(0)
    colf = lax.broadcasted_iota(jnp.int32, (_H, _W), 1).astype(jnp.float32)
    rowf = lax.broadcasted_iota(jnp.int32, (_H, _W), 0).astype(jnp.float32)
    acc = jnp.zeros((_H, _W), jnp.float32)
    for k in range(_BPG):
        bb = bidx * _BPG + k
        gx = tgt_ref[bb, 0] * jnp.float32(_W)
        gy = tgt_ref[bb, 1] * jnp.float32(_H)
        gw = tgt_ref[bb, 2] * jnp.float32(_W)
        gh = tgt_ref[bb, 3] * jnp.float32(_H)
        gxl = gx - gw * 0.5
        gxr = gx + gw * 0.5
        gyl = gy - gh * 0.5
        gyr = gy + gh * 0.5
        g375 = gw * gh * 0.375
        gxlc = gxl - colf
        gxrc = gxr - colf
        gylc = gyl - rowf
        gyrc = gyr - rowf
        for a in range(_A):
            aw, ah = _ANCHORS[a]
            t0 = pred_ref[k, 5 * a + 0]
            t1 = pred_ref[k, 5 * a + 1]
            t2 = pred_ref[k, 5 * a + 2]
            t3 = pred_ref[k, 5 * a + 3]
            t4 = pred_ref[k, 5 * a + 4]
            s0 = 1.0 / (1.0 + jnp.exp(-t0))
            s1 = 1.0 / (1.0 + jnp.exp(-t1))
            pc = 1.0 / (1.0 + jnp.exp(-t4))
            bw2 = jnp.exp(t2) * jnp.float32(aw * 0.5)
            bh2 = jnp.exp(t3) * jnp.float32(ah * 0.5)
            cw = jnp.minimum(s0 + bw2, gxrc) - jnp.maximum(s0 - bw2, gxlc)
            chh = jnp.minimum(s1 + bh2, gyrc) - jnp.maximum(s1 - bh2, gylc)
            carea = jnp.maximum(cw, 0.0) * jnp.maximum(chh, 0.0)
            thr = 1.5 * (bw2 * bh2) + g375
            contrib = jnp.where(carea > thr, 0.0, pc * pc)
            acc = acc + (_sq(s0 - 0.5) + _sq(s1 - 0.5)
                         + t2 * t2 + t3 * t3 + contrib)

    @pl.when(bidx == 0)
    def _():
        acc_ref[...] = jnp.zeros((_H, _W), jnp.float32)

    acc_ref[...] += acc

    @pl.when(bidx == (_B // _BPG) - 1)
    def _():
        out_ref[0, 0] = jnp.sum(acc_ref[...])


_base_tc = pl.pallas_call(
    _base_body,
    grid=(_B // _BPG,),
    in_specs=[
        pl.BlockSpec(memory_space=pltpu.SMEM),
        pl.BlockSpec((_BPG, _C, _H, _W), lambda b: (b, 0, 0, 0)),
    ],
    out_specs=pl.BlockSpec((1, 1), lambda b: (0, 0), memory_space=pltpu.SMEM),
    out_shape=jax.ShapeDtypeStruct((1, 1), jnp.float32),
    scratch_shapes=[pltpu.VMEM((_H, _W), jnp.float32)],
)


def kernel(pred, target, train_out):
    corr = _corr_sc(pred, target)
    base = _base_tc(target, pred)
    loss = (base[0, 0] + jnp.sum(corr)) * 0.5
    return loss + jnp.asarray(train_out, loss.dtype) * 0.0


# speedup vs baseline: 1.0955x; 1.0955x over previous
import functools

import jax
import jax.numpy as jnp
from jax import lax
from jax.experimental import pallas as pl
from jax.experimental.pallas import tpu as pltpu
from jax.experimental.pallas import tpu_sc as plsc

_ANCHORS = [
    (1.3221, 1.73145),
    (3.19275, 4.00944),
    (5.05587, 8.09892),
    (9.47112, 4.84053),
    (11.2364, 10.0071),
]
_OBJECT_SCALE = 5.0
_LN2 = 0.6931471805599453
_B, _C, _H, _W = 16, 25, 32, 32
_A = 5


def _sq(x):
    return x * x


def _log_v(x):
    xi = lax.bitcast_convert_type(x, jnp.int32)
    e = (xi >> 23) - 127
    m = lax.bitcast_convert_type((xi & 0x7FFFFF) | (127 << 23), jnp.float32)
    r = (m - 1.0) / (m + 1.0)
    r2 = r * r
    p = r * (2.0 + r2 * (2.0 / 3.0 + r2 * (2.0 / 5.0
                                           + r2 * (2.0 / 7.0 + r2 * (2.0 / 9.0)))))
    return e.astype(jnp.float32) * _LN2 + p


def _corr_body(pred_hbm, tgt_hbm, out_hbm, cell_v, tgt_v, res_v, dsem):
    c = lax.axis_index("c")
    s = lax.axis_index("s")
    b = s

    @pl.when(c == 0)
    def _():
        pltpu.sync_copy(tgt_hbm, tgt_v)
        lane_i = lax.iota(jnp.int32, 16)
        gv = plsc.load_gather(tgt_v, [jnp.full((16,), b, jnp.int32),
                                      lane_i & 3])

        def pick(off):
            return jnp.sum(jnp.where(lane_i == off, gv, 0.0))

        gx = pick(0) * jnp.float32(_W)
        gy = pick(1) * jnp.float32(_H)
        gw = pick(2) * jnp.float32(_W)
        gh = pick(3) * jnp.float32(_H)
        gi = jnp.clip(gx.astype(jnp.int32), 0, _W - 1)
        gj = jnp.clip(gy.astype(jnp.int32), 0, _H - 1)

        def const_vec(vals):
            v = jnp.full((16,), 1.0, dtype=jnp.float32)
            for idx, val in enumerate(vals):
                v = jnp.where(lane_i == idx, jnp.float32(val), v)
            return v

        awv = const_vec([a[0] for a in _ANCHORS])
        ahv = const_vec([a[1] for a in _ANCHORS])
        inter_v = jnp.minimum(awv, gw) * jnp.minimum(ahv, gh)
        union_v = awv * ahv + gw * gh - inter_v
        ratio_v = jnp.where(lane_i < _A, inter_v / union_v, -1.0)
        best_r = jnp.max(ratio_v)
        hit_v = ratio_v == best_r
        best_a = jnp.min(jnp.where(hit_v, lane_i, jnp.int32(99)))
        aw_b = jnp.sum(jnp.where(lane_i == best_a, awv, 0.0))
        ah_b = jnp.sum(jnp.where(lane_i == best_a, ahv, 0.0))

        cb = 5 * best_a
        pltpu.async_copy(pred_hbm.at[b, pl.ds(cb, 5), gj, :], cell_v,
                         dsem).wait()
        goff = (gi >> 4) << 4
        lane = gi - goff

        def cell_val(k):
            v = cell_v[k, pl.ds(goff, 16)]
            return jnp.full((16,), jnp.sum(jnp.where(lane_i == lane, v, 0.0)))

        t0c = cell_val(0)
        t1c = cell_val(1)
        t2c = cell_val(2)
        t3c = cell_val(3)
        t4c = cell_val(4)
        s0c = 1.0 / (1.0 + jnp.exp(-t0c))
        s1c = 1.0 / (1.0 + jnp.exp(-t1c))
        pcc = 1.0 / (1.0 + jnp.exp(-t4c))
        gif = gi.astype(jnp.float32)
        gjf = gj.astype(jnp.float32)
        gxl = gx - gw * 0.5
        gxr = gx + gw * 0.5
        gyl = gy - gh * 0.5
        gyr = gy + gh * 0.5
        garea = gw * gh
        pbx = s0c + gif
        pby = s1c + gjf
        pbw = jnp.exp(t2c) * aw_b
        pbh = jnp.exp(t3c) * ah_b
        cw = (jnp.minimum(pbx + pbw * 0.5, gxr)
              - jnp.maximum(pbx - pbw * 0.5, gxl))
        chh = (jnp.minimum(pby + pbh * 0.5, gyr)
               - jnp.maximum(pby - pbh * 0.5, gyl))
        carea = jnp.maximum(cw, 0.0) * jnp.maximum(chh, 0.0)
        uarea = pbw * pbh + garea - carea
        tconf = carea / uarea
        mstar = jnp.where(tconf > 0.6, 0.0, 1.0)
        lw = _log_v(jnp.full((16,), gw) / jnp.full((16,), aw_b))
        lh = _log_v(jnp.full((16,), gh) / jnp.full((16,), ah_b))
        delta = (_sq(s0c - (gx - gif)) - _sq(s0c - 0.5)
                 + _sq(s1c - (gy - gjf)) - _sq(s1c - 0.5)
                 + _sq(t2c - lw) - t2c * t2c
                 + _sq(t3c - lh) - t3c * t3c
                 + _OBJECT_SCALE * _sq(pcc - tconf) - mstar * pcc * pcc)
        res_v[...] = jnp.where(lane_i == 0, delta, 0.0)
        pltpu.sync_copy(res_v, out_hbm.at[b])


_corr_sc = functools.partial(
    pl.kernel,
    mesh=plsc.VectorSubcoreMesh(core_axis_name="c", subcore_axis_name="s"),
    out_type=jax.ShapeDtypeStruct((_B, 16), jnp.float32),
    compiler_params=pltpu.CompilerParams(needs_layout_passes=False),
    scratch_types=[
        pltpu.VMEM((5, _W), jnp.float32),
        pltpu.VMEM((_B, 4), jnp.float32),
        pltpu.VMEM((16,), jnp.float32),
        pltpu.SemaphoreType.DMA,
    ],
)(_corr_body)


_BPG = 8


def _base_body(tgt_ref, pred_ref, out_ref, acc_ref):
    bidx = pl.program_id(0)
    colf = lax.broadcasted_iota(jnp.int32, (_H, _W), 1).astype(jnp.float32)
    rowf = lax.broadcasted_iota(jnp.int32, (_H, _W), 0).astype(jnp.float32)
    acc = jnp.zeros((_H, _W), jnp.float32)
    for k in range(_BPG):
        bb = bidx * _BPG + k
        gx = tgt_ref[bb, 0] * jnp.float32(_W)
        gy = tgt_ref[bb, 1] * jnp.float32(_H)
        gw = tgt_ref[bb, 2] * jnp.float32(_W)
        gh = tgt_ref[bb, 3] * jnp.float32(_H)
        gxl = gx - gw * 0.5
        gxr = gx + gw * 0.5
        gyl = gy - gh * 0.5
        gyr = gy + gh * 0.5
        g375 = gw * gh * 0.375
        gxlc = gxl - colf
        gxrc = gxr - colf
        gylc = gyl - rowf
        gyrc = gyr - rowf
        for a in range(_A):
            aw, ah = _ANCHORS[a]
            t0 = pred_ref[k, 5 * a + 0]
            t1 = pred_ref[k, 5 * a + 1]
            t2 = pred_ref[k, 5 * a + 2]
            t3 = pred_ref[k, 5 * a + 3]
            t4 = pred_ref[k, 5 * a + 4]
            s0 = 1.0 / (1.0 + jnp.exp(-t0))
            s1 = 1.0 / (1.0 + jnp.exp(-t1))
            pc = 1.0 / (1.0 + jnp.exp(-t4))
            bw2 = jnp.exp(t2) * jnp.float32(aw * 0.5)
            bh2 = jnp.exp(t3) * jnp.float32(ah * 0.5)
            cw = jnp.minimum(s0 + bw2, gxrc) - jnp.maximum(s0 - bw2, gxlc)
            chh = jnp.minimum(s1 + bh2, gyrc) - jnp.maximum(s1 - bh2, gylc)
            carea = jnp.maximum(cw, 0.0) * jnp.maximum(chh, 0.0)
            thr = 1.5 * (bw2 * bh2) + g375
            contrib = jnp.where(carea > thr, 0.0, pc * pc)
            acc = acc + (_sq(s0 - 0.5) + _sq(s1 - 0.5)
                         + t2 * t2 + t3 * t3 + contrib)

    @pl.when(bidx == 0)
    def _():
        acc_ref[...] = jnp.zeros((_H, _W), jnp.float32)

    acc_ref[...] += acc

    @pl.when(bidx == (_B // _BPG) - 1)
    def _():
        out_ref[0, 0] = jnp.sum(acc_ref[...])


_base_tc = pl.pallas_call(
    _base_body,
    grid=(_B // _BPG,),
    in_specs=[
        pl.BlockSpec(memory_space=pltpu.SMEM),
        pl.BlockSpec((_BPG, _C, _H, _W), lambda b: (b, 0, 0, 0)),
    ],
    out_specs=pl.BlockSpec((1, 1), lambda b: (0, 0), memory_space=pltpu.SMEM),
    out_shape=jax.ShapeDtypeStruct((1, 1), jnp.float32),
    scratch_shapes=[pltpu.VMEM((_H, _W), jnp.float32)],
)


def kernel(pred, target, train_out):
    corr = _corr_sc(pred, target)
    base = _base_tc(target, pred)
    loss = (base[0, 0] + jnp.sum(corr)) * 0.5
    return loss + jnp.asarray(train_out, loss.dtype) * 0.0
